# Initial kernel scaffold; baseline (speedup 1.0000x reference)
import jax
import jax.numpy as jnp
from jax.experimental import pallas as pl


def kernel(x, edge_index, W1, a_src1, a_dst1, b1, W2, a_src2, a_dst2, b2, lw1, lb1, lw2, lb2):
    raise NotImplementedError("write your pallas kernel here")


# speedup vs baseline: 32.5592x; 32.5592x over previous
import functools

import jax
import jax.numpy as jnp
from jax import lax
from jax.experimental import pallas as pl
from jax.experimental.pallas import tpu as pltpu
from jax.experimental.pallas import tpu_sc as plsc

N = 10000
D = 128
HID = 64
E = 320000
NCORE = 2
NSUB = 16
HALF = D // NCORE
EPT = E // NSUB
CH = 80
NCHUNK = EPT // CH
NBUF = 3
RPT = 640
N_PAD = NSUB * RPT
NVB = RPT // 80

BN = N


def _tc_embed_body(pre_bias, inp_ref, w_ref, a_ref, b_ref, hp_ref, att_ref):
    if pre_bias:
        z = jnp.concatenate([inp_ref[0], inp_ref[1]], axis=1)
        z = jnp.maximum(z + b_ref[...], 0.0)
    else:
        z = inp_ref[...].reshape(BN, D)
    h = jnp.dot(z, w_ref[...], preferred_element_type=jnp.float32)
    hp_ref[...] = h.reshape(BN, NCORE, HALF)
    att_ref[...] = lax.dot_general(
        a_ref[...], h, (((1,), (1,)), ((), ())),
        preferred_element_type=jnp.float32)


def _tc_embed(inp, W, A, b, pre_bias):
    if inp.ndim == 2:
        in_spec = pl.BlockSpec((BN, D), lambda i: (0, 0))
    else:
        in_spec = pl.BlockSpec((NCORE, BN, HALF), lambda i: (0, 0, 0))
    return pl.pallas_call(
        functools.partial(_tc_embed_body, pre_bias),
        grid=(1,),
        in_specs=[
            in_spec,
            pl.BlockSpec((D, D), lambda i: (0, 0)),
            pl.BlockSpec((NCORE, D), lambda i: (0, 0)),
            pl.BlockSpec((1, D), lambda i: (0, 0)),
        ],
        out_specs=[
            pl.BlockSpec((BN, NCORE, HALF), lambda i: (0, 0, 0)),
            pl.BlockSpec((NCORE, BN), lambda i: (0, 0)),
        ],
        out_shape=[
            jax.ShapeDtypeStruct((N, NCORE, HALF), jnp.float32),
            jax.ShapeDtypeStruct((NCORE, N), jnp.float32),
        ],
    )(inp, W, A, b)


def _tc_head_body(o_ref, b2_ref, w1_ref, b1_ref, w2_ref, bs_ref, out_ref):
    z = jnp.concatenate([o_ref[0], o_ref[1]], axis=1) + b2_ref[...]
    hh = jnp.maximum(
        jnp.dot(z, w1_ref[...], preferred_element_type=jnp.float32) + b1_ref[...],
        0.0,
    )
    out_ref[...] = (
        jnp.dot(hh, w2_ref[...], preferred_element_type=jnp.float32) + bs_ref[...]
    )


def _tc_head(o2, b2, lw1, lb1, lw2, lb2):
    return pl.pallas_call(
        _tc_head_body,
        grid=(1,),
        in_specs=[
            pl.BlockSpec((NCORE, BN, HALF), lambda i: (0, 0, 0)),
            pl.BlockSpec((1, D), lambda i: (0, 0)),
            pl.BlockSpec((D, HID), lambda i: (0, 0)),
            pl.BlockSpec((1, HID), lambda i: (0, 0)),
            pl.BlockSpec((HID, 1), lambda i: (0, 0)),
            pl.BlockSpec((1, 1), lambda i: (0, 0)),
        ],
        out_specs=pl.BlockSpec((BN, 1), lambda i: (0, 0)),
        out_shape=jax.ShapeDtypeStruct((N, 1), jnp.float32),
    )(o2, b2, lw1, lb1, lw2, lb2)


def _sc_body(src_hbm, dst_hbm, att_hbm, hp_hbm, out_hbm,
             idx_v, dst_v, as_v, ad_v, ldenom_v, rows_v, exbuf_v,
             dtmp_v, dacc_v, acc_sp, parts_sp, gsem, ssem):
    cid = lax.axis_index("c")
    sid = lax.axis_index("s")
    zero16 = jnp.zeros((16,), jnp.float32)
    iz16 = jnp.zeros((16,), jnp.int32)

    pltpu.sync_copy(src_hbm.at[sid], idx_v)
    pltpu.sync_copy(dst_hbm.at[sid], dst_v)
    pltpu.sync_copy(att_hbm.at[0], as_v)
    pltpu.sync_copy(att_hbm.at[1], ad_v)

    def _z(k, _):
        ldenom_v[pl.ds(k * 16, 16)] = zero16
        return _
    lax.fori_loop(0, N_PAD // 16, _z, None)

    def _zr(r, _):
        for f in range(HALF // 16):
            rows_v[0, r, pl.ds(f * 16, 16)] = zero16
        return _
    lax.fori_loop(0, CH, _zr, None)

    for i in range(NVB):
        row0 = sid * RPT + i * CH

        @pl.when(row0 < N)
        def _():
            pltpu.sync_copy(rows_v.at[0], acc_sp.at[pl.ds(row0, CH)])

    def _pa(k, _):
        r = k // 5
        g = k - r * 5
        s16 = idx_v[r, pl.ds(g * 16, 16)]
        d16 = dst_v[r, pl.ds(g * 16, 16)]
        a_s = plsc.load_gather(as_v, [s16])
        a_d = plsc.load_gather(ad_v, [d16])
        e = a_s + a_d
        e = jnp.where(e >= 0.0, e, 0.2 * e)
        ex = jnp.exp(e)
        plsc.addupdate_scatter(ldenom_v, [d16], ex)
        idx_v[r, pl.ds(g * 16, 16)] = s16 * 2 + cid
        return _
    lax.fori_loop(0, EPT // 16, _pa, None)

    pltpu.sync_copy(ldenom_v, parts_sp.at[sid])
    plsc.subcore_barrier()

    def _zd(k, _):
        dacc_v[pl.ds(k * 16, 16)] = zero16
        return _
    lax.fori_loop(0, RPT // 16, _zd, None)
    for p in range(NSUB):
        pltpu.sync_copy(parts_sp.at[p, pl.ds(sid * RPT, RPT)], dtmp_v)

        def _acc(k, _):
            sl = pl.ds(k * 16, 16)
            dacc_v[sl] = dacc_v[sl] + dtmp_v[sl]
            return _
        lax.fori_loop(0, RPT // 16, _acc, None)

    def _rec(k, _):
        sl = pl.ds(k * 16, 16)
        dacc_v[sl] = 1.0 / (dacc_v[sl] + 1e-16)
        return _
    lax.fori_loop(0, RPT // 16, _rec, None)

    def _gather(c, b):
        pltpu.async_copy(hp_hbm.at[idx_v.at[c]], rows_v.at[b], gsem.at[b])

    def _gwait(b):
        pltpu.make_async_copy(hp_hbm.at[idx_v.at[0]], rows_v.at[b],
                              gsem.at[b]).wait()

    def _scatter(c, b):
        pltpu.async_copy(rows_v.at[b], acc_sp.at[dst_v.at[c]], ssem.at[b],
                         add=True)

    def _swait(b):
        pltpu.make_async_copy(rows_v.at[b], acc_sp.at[dst_v.at[0]],
                              ssem.at[b]).wait()

    _gather(0, 0)
    _gather(1, 1)

    def _pb(t, _):
        for b in range(NBUF):
            c = t * NBUF + b

            @pl.when(c < NCHUNK)
            def _():
                _gwait(b)

                def _scale_grp(g, _):
                    i16 = idx_v[c, pl.ds(g * 16, 16)]
                    d16 = dst_v[c, pl.ds(g * 16, 16)]
                    s16 = (i16 - cid) >> 1
                    a_s = plsc.load_gather(as_v, [s16])
                    a_d = plsc.load_gather(ad_v, [d16])
                    e = a_s + a_d
                    e = jnp.where(e >= 0.0, e, 0.2 * e)
                    exbuf_v[...] = jnp.exp(e)

                    def _row(r2, _):
                        w = plsc.load_gather(exbuf_v, [iz16 + r2])
                        row = g * 16 + r2
                        for f in range(HALF // 16):
                            sl = pl.ds(f * 16, 16)
                            rows_v[b, row, sl] = rows_v[b, row, sl] * w
                        return _
                    lax.fori_loop(0, 16, _row, None)
                    return _
                lax.fori_loop(0, CH // 16, _scale_grp, None)
                _scatter(c, b)

                @pl.when(c + 2 < NCHUNK)
                def _():
                    b2 = (b + 2) % NBUF

                    @pl.when(c >= 1)
                    def _():
                        _swait(b2)
                    _gather(c + 2, b2)
        return _
    lax.fori_loop(0, (NCHUNK + NBUF - 1) // NBUF, _pb, None)
    _swait(1)
    _swait(2)
    _swait(0)
    plsc.subcore_barrier()

    for j in range(NVB):
        row0 = sid * RPT + j * CH

        @pl.when(row0 < N)
        def _():
            pltpu.sync_copy(acc_sp.at[pl.ds(row0, CH)], rows_v.at[0])

            def _div(r, _):
                w = plsc.load_gather(dacc_v, [iz16 + (j * CH + r)])
                for f in range(HALF // 16):
                    sl = pl.ds(f * 16, 16)
                    rows_v[0, r, sl] = rows_v[0, r, sl] * w
                return _
            lax.fori_loop(0, CH, _div, None)
            pltpu.sync_copy(rows_v.at[0], out_hbm.at[cid, pl.ds(row0, CH)])


def _sc_gat_edges(src4, dst4, att, hp_flat):
    mesh = plsc.VectorSubcoreMesh(core_axis_name="c", subcore_axis_name="s")
    return pl.kernel(
        _sc_body,
        out_type=jax.ShapeDtypeStruct((NCORE, N, HALF), jnp.float32),
        mesh=mesh,
        compiler_params=pltpu.CompilerParams(
            needs_layout_passes=False, use_tc_tiling_on_sc=False),
        scratch_types=[
            pltpu.VMEM((NCHUNK, CH), jnp.int32),
            pltpu.VMEM((NCHUNK, CH), jnp.int32),
            pltpu.VMEM((N,), jnp.float32),
            pltpu.VMEM((N,), jnp.float32),
            pltpu.VMEM((N_PAD,), jnp.float32),
            pltpu.VMEM((NBUF, CH, HALF), jnp.float32),
            pltpu.VMEM((16,), jnp.float32),
            pltpu.VMEM((RPT,), jnp.float32),
            pltpu.VMEM((RPT,), jnp.float32),
            pltpu.VMEM_SHARED((N, HALF), jnp.float32),
            pltpu.HBM((NSUB, N_PAD), jnp.float32),
            pltpu.SemaphoreType.DMA((NBUF,)),
            pltpu.SemaphoreType.DMA((NBUF,)),
        ],
    )(src4, dst4, att, hp_flat)


def kernel(x, edge_index, W1, a_src1, a_dst1, b1, W2, a_src2, a_dst2, b2,
           lw1, lb1, lw2, lb2):
    src = edge_index[0].astype(jnp.int32)
    dst = edge_index[1].astype(jnp.int32)
    src4 = src.reshape(NSUB, NCHUNK, CH)
    dst4 = dst.reshape(NSUB, NCHUNK, CH)

    A1 = jnp.concatenate([a_src1, a_dst1], axis=0)
    A2 = jnp.concatenate([a_src2, a_dst2], axis=0)
    b1r = b1.reshape(1, D)
    b2r = b2.reshape(1, D)

    hp1, att1 = _tc_embed(x, W1, A1, b1r, pre_bias=False)
    o1 = _sc_gat_edges(src4, dst4, att1, hp1.reshape(NCORE * N, HALF))
    hp2, att2 = _tc_embed(o1, W2, A2, b1r, pre_bias=True)
    o2 = _sc_gat_edges(src4, dst4, att2, hp2.reshape(NCORE * N, HALF))
    return _tc_head(o2, b2r, lw1, lb1.reshape(1, HID), lw2,
                    lb2.reshape(1, 1))
